# Initial kernel scaffold; baseline (speedup 1.0000x reference)
import jax
import jax.numpy as jnp
from jax.experimental import pallas as pl


def kernel(feat0, feat1, neighbor_indices, neighbor_mask, rel_dist, basis, params):
    raise NotImplementedError("write your pallas kernel here")


# speedup vs baseline: 1.6057x; 1.6057x over previous
import jax
import jax.numpy as jnp
from jax.experimental import pallas as pl

PREC = jax.lax.Precision.HIGHEST

DEG_PAIRS = ((0, 0), (0, 1), (1, 0), (1, 1))
N = 256
NB = 16
F = 16
HEADS = 2
DH = 16
HID = HEADS * DH
MID = 128
BN = 16
E = BN * NB

W3_WIDTHS = (512, 512, 512, 1536)
W3_OFFS = (0, 512, 1024, 1536, 3072, 3584, 4096, 4608)


def _ln(x, g, b):
    mu = jnp.mean(x, axis=-1, keepdims=True)
    var = jnp.mean((x - mu) ** 2, axis=-1, keepdims=True)
    return (x - mu) / jnp.sqrt(var + 1e-5) * g + b


def _iota(shape, dim):
    return jax.lax.broadcasted_iota(jnp.int32, shape, dim)


def _fused_kernel(idx_ref, xtab_ref, rel_ref, b00_ref, b01_ref, b10_ref, b11_ref,
                  madd_ref, f0_ref, f1_ref,
                  w1_ref, b1_ref, g1_ref, be1_ref,
                  w2_ref, b2_ref, g2_ref, be2_ref,
                  w3_ref, b3_ref, wq0_ref, wq1_ref, wo0_ref, wo1_ref,
                  out0_ref, out1_ref):
    f32 = jnp.float32
    idx = idx_ref[...]
    onehot = (idx == _iota((E, N), 1)).astype(f32)
    xg = jnp.dot(onehot, xtab_ref[...], preferred_element_type=f32, precision=PREC)
    xg0 = xg[:, :F]
    xg1 = [xg[:, F + m * F:F + (m + 1) * F] for m in range(3)]

    rel = rel_ref[...]
    y2s = []
    for p in range(8):
        y = rel * w1_ref[p:p + 1, :] + b1_ref[p:p + 1, :]
        y = jax.nn.relu(_ln(y, g1_ref[p:p + 1, :], be1_ref[p:p + 1, :]))
        y = jnp.dot(y.astype(jnp.bfloat16), w2_ref[p],
                    preferred_element_type=f32) + b2_ref[p:p + 1, :]
        y = jax.nn.relu(_ln(y, g2_ref[p:p + 1, :], be2_ref[p:p + 1, :]))
        y2s.append(y.astype(jnp.bfloat16))

    S16 = (_iota((512, HID), 0) // F == _iota((512, HID), 1)).astype(f32)
    S48 = (_iota((1536, HID), 0) // 48 == _iota((1536, HID), 1)).astype(f32)

    b00 = b00_ref[...]
    b01 = b01_ref[...]
    b10 = b10_ref[...]
    b11 = b11_ref[...]

    kv0 = []
    kv1 = []
    for cv in range(2):
        acc0 = jnp.zeros((E, HID), f32)
        acc1 = [jnp.zeros((E, HID), f32) for _ in range(3)]
        for ip, (di, do) in enumerate(DEG_PAIRS):
            p = cv * 4 + ip
            w3 = w3_ref[:, W3_OFFS[p]:W3_OFFS[p] + W3_WIDTHS[ip]]
            b3 = b3_ref[:, W3_OFFS[p]:W3_OFFS[p] + W3_WIDTHS[ip]]
            y3 = jnp.dot(y2s[p], w3, preferred_element_type=f32) + b3
            if (di, do) == (0, 0):
                xb = b00 * xg0
                prod = y3 * jnp.concatenate([xb] * HID, axis=1)
                acc0 = acc0 + jnp.dot(prod, S16, preferred_element_type=f32, precision=PREC)
            elif (di, do) == (0, 1):
                for mo in range(3):
                    xb = b01[:, mo:mo + 1] * xg0
                    prod = y3 * jnp.concatenate([xb] * HID, axis=1)
                    acc1[mo] = acc1[mo] + jnp.dot(prod, S16, preferred_element_type=f32, precision=PREC)
            elif (di, do) == (1, 0):
                xb = jnp.zeros((E, F), f32)
                for mi in range(3):
                    xb = xb + xg1[mi] * b10[:, mi:mi + 1]
                prod = y3 * jnp.concatenate([xb] * HID, axis=1)
                acc0 = acc0 + jnp.dot(prod, S16, preferred_element_type=f32, precision=PREC)
            else:
                for mo in range(3):
                    parts = []
                    for c in range(3):
                        xc = jnp.zeros((E, F), f32)
                        for mi in range(3):
                            col = 9 * mo + 3 * mi + c
                            xc = xc + xg1[mi] * b11[:, col:col + 1]
                        parts.append(xc)
                    xb = jnp.concatenate(parts, axis=1)
                    prod = y3 * jnp.concatenate([xb] * HID, axis=1)
                    acc1[mo] = acc1[mo] + jnp.dot(prod, S48, preferred_element_type=f32, precision=PREC)
        kv0.append(acc0)
        kv1.append(acc1)
    K0, V0 = kv0
    K1, V1 = kv1

    scale = DH ** -0.5
    madd = madd_ref[...]
    f0 = f0_ref[...]
    f1 = f1_ref[...]
    q0 = jnp.dot(f0, wq0_ref[...], preferred_element_type=f32, precision=PREC)
    q1 = [jnp.dot(f1[:, m * F:(m + 1) * F], wq1_ref[...], preferred_element_type=f32, precision=PREC)
          for m in range(3)]

    Sh = (_iota((HID, HEADS), 0) // DH == _iota((HID, HEADS), 1)).astype(f32)
    Gn = (_iota((BN, E), 1) // NB == _iota((BN, E), 0)).astype(f32)
    GnT = (_iota((E, BN), 0) // NB == _iota((E, BN), 1)).astype(f32)
    Gk = (_iota((E, NB), 0) % NB == _iota((E, NB), 1)).astype(f32)
    qe0 = jnp.dot(GnT, q0, preferred_element_type=f32, precision=PREC)
    qe1 = [jnp.dot(GnT, q, preferred_element_type=f32, precision=PREC) for q in q1]

    def softmax_full(s):
        sim = jnp.dot(s, Sh, preferred_element_type=f32, precision=PREC) * scale + madd
        cols = []
        for h in range(HEADS):
            mat = jnp.dot(Gn, sim[:, h:h + 1] * Gk, preferred_element_type=f32, precision=PREC)
            mx = mat.max(axis=-1, keepdims=True)
            ex = jnp.exp(mat - mx)
            at = ex / ex.sum(axis=-1, keepdims=True)
            ate = (jnp.dot(GnT, at, preferred_element_type=f32, precision=PREC) * Gk).sum(
                axis=-1, keepdims=True)
            cols.append(jnp.broadcast_to(ate, (E, DH)))
        return jnp.concatenate(cols, axis=1)

    a0 = softmax_full(qe0 * K0)
    o0 = jnp.dot(Gn, a0 * V0, preferred_element_type=f32, precision=PREC)
    out0_ref[...] = jnp.dot(o0, wo0_ref[...], preferred_element_type=f32, precision=PREC)

    s1 = qe1[0] * K1[0] + qe1[1] * K1[1] + qe1[2] * K1[2]
    a1 = softmax_full(s1)
    outs = []
    for m in range(3):
        o1 = jnp.dot(Gn, a1 * V1[m], preferred_element_type=f32, precision=PREC)
        outs.append(jnp.dot(o1, wo1_ref[...], preferred_element_type=f32, precision=PREC))
    out1_ref[...] = jnp.concatenate(outs, axis=1)


def kernel(feat0, feat1, neighbor_indices, neighbor_mask, rel_dist, basis, params):
    B = feat0.shape[0]
    f0 = feat0.reshape(N, F).astype(jnp.float32)
    f1m = feat1.reshape(N, F, 3).transpose(0, 2, 1).reshape(N, 3 * F).astype(jnp.float32)
    xtab = jnp.concatenate([f0, f1m], axis=1)
    idx = neighbor_indices.reshape(N * NB, 1).astype(jnp.int32)
    rel = rel_dist.reshape(N * NB, 1).astype(jnp.float32)
    madd = jnp.where(neighbor_mask.reshape(N * NB, 1), 0.0, -1e30).astype(jnp.float32)
    b00 = basis['0,0'].reshape(N * NB, 1)
    b01 = basis['0,1'].reshape(N * NB, 3)
    b10 = basis['1,0'].reshape(N * NB, 3)
    b11 = basis['1,1'].reshape(N * NB, 27)

    pk, pv = params['to_k'], params['to_v']
    pipes = [pk['0,0'], pk['0,1'], pk['1,0'], pk['1,1'],
             pv['0,0'], pv['0,1'], pv['1,0'], pv['1,1']]

    def permute_w3(p, ip):
        w3, b3 = p['w3'], p['b3']
        if ip == 3:
            w3 = w3.reshape(MID, HID, F, 3).transpose(0, 1, 3, 2).reshape(MID, 1536)
            b3 = b3.reshape(HID, F, 3).transpose(0, 2, 1).reshape(1536)
        return w3, b3

    w3b3 = [permute_w3(p, i % 4) for i, p in enumerate(pipes)]
    w1 = jnp.stack([p['w1'].reshape(MID) for p in pipes])
    b1 = jnp.stack([p['b1'] for p in pipes])
    g1 = jnp.stack([p['g1'] for p in pipes])
    be1 = jnp.stack([p['be1'] for p in pipes])
    w2 = jnp.stack([p['w2'] for p in pipes]).astype(jnp.bfloat16)
    b2 = jnp.stack([p['b2'] for p in pipes])
    g2 = jnp.stack([p['g2'] for p in pipes])
    be2 = jnp.stack([p['be2'] for p in pipes])
    w3 = jnp.concatenate([w for w, _ in w3b3], axis=1).astype(jnp.bfloat16)
    b3 = jnp.concatenate([b for _, b in w3b3])[None, :]
    wq0 = params['to_q']['0']
    wq1 = params['to_q']['1']
    wo0 = params['to_out']['0']
    wo1 = params['to_out']['1']

    def blk(shape):
        return pl.BlockSpec(shape, lambda i: (i,) + (0,) * (len(shape) - 1))

    def rep(shape):
        return pl.BlockSpec(shape, lambda i: (0,) * len(shape))

    out0, out1 = pl.pallas_call(
        _fused_kernel,
        grid=(N // BN,),
        in_specs=[
            blk((E, 1)),
            rep((N, 4 * F)),
            blk((E, 1)),
            blk((E, 1)),
            blk((E, 3)),
            blk((E, 3)),
            blk((E, 27)),
            blk((E, 1)),
            blk((BN, F)),
            blk((BN, 3 * F)),
            rep((8, MID)), rep((8, MID)), rep((8, MID)), rep((8, MID)),
            rep((8, MID, MID)),
            rep((8, MID)), rep((8, MID)), rep((8, MID)),
            rep((MID, 6144)), rep((1, 6144)),
            rep((F, HID)), rep((F, HID)), rep((HID, F)), rep((HID, F)),
        ],
        out_specs=[blk((BN, F)), blk((BN, 3 * F))],
        out_shape=[jax.ShapeDtypeStruct((N, F), jnp.float32),
                   jax.ShapeDtypeStruct((N, 3 * F), jnp.float32)],
    )(idx, xtab, rel, b00, b01, b10, b11, madd, f0, f1m,
      w1, b1, g1, be1, w2, b2, g2, be2, w3, b3, wq0, wq1, wo0, wo1)

    out1 = out1.reshape(N, 3, F).transpose(0, 2, 1)
    return out0.reshape(B, N, F, 1), out1.reshape(B, N, F, 3)


# speedup vs baseline: 2.8735x; 1.7895x over previous
import jax
import jax.numpy as jnp
from jax.experimental import pallas as pl

PREC = jax.lax.Precision.HIGHEST

DEG_PAIRS = ((0, 0), (0, 1), (1, 0), (1, 1))
N = 256
NB = 16
F = 16
HEADS = 2
DH = 16
HID = HEADS * DH
MID = 128
BN = 16
E = BN * NB

W3_WIDTHS = (512, 512, 512, 1536)
W3_OFFS = (0, 512, 1024, 1536, 3072, 3584, 4096, 4608)


def _ln(x, g, b):
    mu = jnp.mean(x, axis=-1, keepdims=True)
    var = jnp.mean((x - mu) ** 2, axis=-1, keepdims=True)
    return (x - mu) / jnp.sqrt(var + 1e-5) * g + b


def _iota(shape, dim):
    return jax.lax.broadcasted_iota(jnp.int32, shape, dim)


def _fused_kernel(idx_ref, xtab_ref, rel_ref, b00_ref, b01_ref, b10_ref, b11_ref,
                  madd_ref, f0_ref, f1_ref,
                  w1_ref, b1_ref, g1_ref, be1_ref,
                  w2_ref, b2_ref, g2_ref, be2_ref,
                  w3_ref, b3_ref, wq0_ref, wq1_ref, wo0_ref, wo1_ref,
                  out0_ref, out1_ref):
    f32 = jnp.float32
    idx = idx_ref[...]
    onehot = (idx == _iota((E, N), 1)).astype(f32)
    xg = jnp.dot(onehot, xtab_ref[...], preferred_element_type=f32, precision=PREC)
    xg0 = xg[:, :F]
    xg1 = [xg[:, F + m * F:F + (m + 1) * F] for m in range(3)]

    rel = rel_ref[...]
    y2s = []
    for p in range(8):
        y = rel * w1_ref[p:p + 1, :] + b1_ref[p:p + 1, :]
        y = jax.nn.relu(_ln(y, g1_ref[p:p + 1, :], be1_ref[p:p + 1, :]))
        y = jnp.dot(y.astype(jnp.bfloat16), w2_ref[p],
                    preferred_element_type=f32) + b2_ref[p:p + 1, :]
        y = jax.nn.relu(_ln(y, g2_ref[p:p + 1, :], be2_ref[p:p + 1, :]))
        y2s.append(y.astype(jnp.bfloat16))

    Ttile = (_iota((F, F * HID), 1) // HID == _iota((F, F * HID), 0)).astype(f32)
    xg0t = jnp.dot(xg0, Ttile, preferred_element_type=f32, precision=PREC)
    xg1t = [jnp.dot(xg1[m], Ttile, preferred_element_type=f32, precision=PREC)
            for m in range(3)]

    def foldsum(prod, fc):
        w = fc
        while w % 2 == 0:
            half = (w // 2) * HID
            prod = prod[:, :half] + prod[:, half:]
            w //= 2
        if w == 3:
            prod = prod[:, :HID] + prod[:, HID:2 * HID] + prod[:, 2 * HID:]
        return prod

    b00 = b00_ref[...]
    b01 = b01_ref[...]
    b10 = b10_ref[...]
    b11 = b11_ref[...]

    kv0 = []
    kv1 = []
    for cv in range(2):
        acc0 = jnp.zeros((E, HID), f32)
        acc1 = [jnp.zeros((E, HID), f32) for _ in range(3)]
        for ip, (di, do) in enumerate(DEG_PAIRS):
            p = cv * 4 + ip
            w3 = w3_ref[:, W3_OFFS[p]:W3_OFFS[p] + W3_WIDTHS[ip]]
            b3 = b3_ref[:, W3_OFFS[p]:W3_OFFS[p] + W3_WIDTHS[ip]]
            y3 = jnp.dot(y2s[p], w3, preferred_element_type=f32) + b3
            if (di, do) == (0, 0):
                acc0 = acc0 + foldsum(y3 * (b00 * xg0t), F)
            elif (di, do) == (0, 1):
                for mo in range(3):
                    acc1[mo] = acc1[mo] + foldsum(y3 * (b01[:, mo:mo + 1] * xg0t), F)
            elif (di, do) == (1, 0):
                xbt = (b10[:, 0:1] * xg1t[0] + b10[:, 1:2] * xg1t[1]
                       + b10[:, 2:3] * xg1t[2])
                acc0 = acc0 + foldsum(y3 * xbt, F)
            else:
                for mo in range(3):
                    parts = []
                    for c in range(3):
                        xc = jnp.zeros((E, F * HID), f32)
                        for mi in range(3):
                            col = 9 * mo + 3 * mi + c
                            xc = xc + b11[:, col:col + 1] * xg1t[mi]
                        parts.append(xc)
                    xbt = jnp.concatenate(parts, axis=1)
                    acc1[mo] = acc1[mo] + foldsum(y3 * xbt, 48)
        kv0.append(acc0)
        kv1.append(acc1)
    K0, V0 = kv0
    K1, V1 = kv1

    scale = DH ** -0.5
    madd = madd_ref[...]
    f0 = f0_ref[...]
    f1 = f1_ref[...]
    q0 = jnp.dot(f0, wq0_ref[...], preferred_element_type=f32, precision=PREC)
    q1 = [jnp.dot(f1[:, m * F:(m + 1) * F], wq1_ref[...], preferred_element_type=f32, precision=PREC)
          for m in range(3)]

    Sh = (_iota((HID, HEADS), 0) // DH == _iota((HID, HEADS), 1)).astype(f32)
    Gn = (_iota((BN, E), 1) // NB == _iota((BN, E), 0)).astype(f32)
    GnT = (_iota((E, BN), 0) // NB == _iota((E, BN), 1)).astype(f32)
    Gk = (_iota((E, NB), 0) % NB == _iota((E, NB), 1)).astype(f32)
    qe0 = jnp.dot(GnT, q0, preferred_element_type=f32, precision=PREC)
    qe1 = [jnp.dot(GnT, q, preferred_element_type=f32, precision=PREC) for q in q1]

    def softmax_full(s):
        sim = jnp.dot(s, Sh, preferred_element_type=f32, precision=PREC) * scale + madd
        cols = []
        for h in range(HEADS):
            mat = jnp.dot(Gn, sim[:, h:h + 1] * Gk, preferred_element_type=f32, precision=PREC)
            mx = mat.max(axis=-1, keepdims=True)
            ex = jnp.exp(mat - mx)
            at = ex / ex.sum(axis=-1, keepdims=True)
            ate = (jnp.dot(GnT, at, preferred_element_type=f32, precision=PREC) * Gk).sum(
                axis=-1, keepdims=True)
            cols.append(jnp.broadcast_to(ate, (E, DH)))
        return jnp.concatenate(cols, axis=1)

    a0 = softmax_full(qe0 * K0)
    o0 = jnp.dot(Gn, a0 * V0, preferred_element_type=f32, precision=PREC)
    out0_ref[...] = jnp.dot(o0, wo0_ref[...], preferred_element_type=f32, precision=PREC)

    s1 = qe1[0] * K1[0] + qe1[1] * K1[1] + qe1[2] * K1[2]
    a1 = softmax_full(s1)
    outs = []
    for m in range(3):
        o1 = jnp.dot(Gn, a1 * V1[m], preferred_element_type=f32, precision=PREC)
        outs.append(jnp.dot(o1, wo1_ref[...], preferred_element_type=f32, precision=PREC))
    out1_ref[...] = jnp.concatenate(outs, axis=1)


def kernel(feat0, feat1, neighbor_indices, neighbor_mask, rel_dist, basis, params):
    B = feat0.shape[0]
    f0 = feat0.reshape(N, F).astype(jnp.float32)
    f1m = feat1.reshape(N, F, 3).transpose(0, 2, 1).reshape(N, 3 * F).astype(jnp.float32)
    xtab = jnp.concatenate([f0, f1m], axis=1)
    idx = neighbor_indices.reshape(N * NB, 1).astype(jnp.int32)
    rel = rel_dist.reshape(N * NB, 1).astype(jnp.float32)
    madd = jnp.where(neighbor_mask.reshape(N * NB, 1), 0.0, -1e30).astype(jnp.float32)
    b00 = basis['0,0'].reshape(N * NB, 1)
    b01 = basis['0,1'].reshape(N * NB, 3)
    b10 = basis['1,0'].reshape(N * NB, 3)
    b11 = basis['1,1'].reshape(N * NB, 27)

    pk, pv = params['to_k'], params['to_v']
    pipes = [pk['0,0'], pk['0,1'], pk['1,0'], pk['1,1'],
             pv['0,0'], pv['0,1'], pv['1,0'], pv['1,1']]

    def permute_w3(p, ip):
        w3, b3 = p['w3'], p['b3']
        if ip == 3:
            w3 = w3.reshape(MID, HID, F, 3).transpose(0, 3, 2, 1).reshape(MID, 1536)
            b3 = b3.reshape(HID, F, 3).transpose(2, 1, 0).reshape(1536)
        else:
            w3 = w3.reshape(MID, HID, F).transpose(0, 2, 1).reshape(MID, 512)
            b3 = b3.reshape(HID, F).transpose(1, 0).reshape(512)
        return w3, b3

    w3b3 = [permute_w3(p, i % 4) for i, p in enumerate(pipes)]
    w1 = jnp.stack([p['w1'].reshape(MID) for p in pipes])
    b1 = jnp.stack([p['b1'] for p in pipes])
    g1 = jnp.stack([p['g1'] for p in pipes])
    be1 = jnp.stack([p['be1'] for p in pipes])
    w2 = jnp.stack([p['w2'] for p in pipes]).astype(jnp.bfloat16)
    b2 = jnp.stack([p['b2'] for p in pipes])
    g2 = jnp.stack([p['g2'] for p in pipes])
    be2 = jnp.stack([p['be2'] for p in pipes])
    w3 = jnp.concatenate([w for w, _ in w3b3], axis=1).astype(jnp.bfloat16)
    b3 = jnp.concatenate([b for _, b in w3b3])[None, :]
    wq0 = params['to_q']['0']
    wq1 = params['to_q']['1']
    wo0 = params['to_out']['0']
    wo1 = params['to_out']['1']

    def blk(shape):
        return pl.BlockSpec(shape, lambda i: (i,) + (0,) * (len(shape) - 1))

    def rep(shape):
        return pl.BlockSpec(shape, lambda i: (0,) * len(shape))

    out0, out1 = pl.pallas_call(
        _fused_kernel,
        grid=(N // BN,),
        in_specs=[
            blk((E, 1)),
            rep((N, 4 * F)),
            blk((E, 1)),
            blk((E, 1)),
            blk((E, 3)),
            blk((E, 3)),
            blk((E, 27)),
            blk((E, 1)),
            blk((BN, F)),
            blk((BN, 3 * F)),
            rep((8, MID)), rep((8, MID)), rep((8, MID)), rep((8, MID)),
            rep((8, MID, MID)),
            rep((8, MID)), rep((8, MID)), rep((8, MID)),
            rep((MID, 6144)), rep((1, 6144)),
            rep((F, HID)), rep((F, HID)), rep((HID, F)), rep((HID, F)),
        ],
        out_specs=[blk((BN, F)), blk((BN, 3 * F))],
        out_shape=[jax.ShapeDtypeStruct((N, F), jnp.float32),
                   jax.ShapeDtypeStruct((N, 3 * F), jnp.float32)],
    )(idx, xtab, rel, b00, b01, b10, b11, madd, f0, f1m,
      w1, b1, g1, be1, w2, b2, g2, be2, w3, b3, wq0, wq1, wo0, wo1)

    out1 = out1.reshape(N, 3, F).transpose(0, 2, 1)
    return out0.reshape(B, N, F, 1), out1.reshape(B, N, F, 3)


# speedup vs baseline: 2.9019x; 1.0099x over previous
import jax
import jax.numpy as jnp
from jax.experimental import pallas as pl

PREC = jax.lax.Precision.HIGHEST

DEG_PAIRS = ((0, 0), (0, 1), (1, 0), (1, 1))
N = 256
NB = 16
F = 16
HEADS = 2
DH = 16
HID = HEADS * DH
MID = 128
BN = 16
E = BN * NB

W3_WIDTHS = (512, 512, 512, 1536)
W3_OFFS = (0, 512, 1024, 1536, 3072, 3584, 4096, 4608)


def _ln(x, g, b):
    mu = jnp.mean(x, axis=-1, keepdims=True)
    var = jnp.mean((x - mu) ** 2, axis=-1, keepdims=True)
    return (x - mu) / jnp.sqrt(var + 1e-5) * g + b


def _iota(shape, dim):
    return jax.lax.broadcasted_iota(jnp.int32, shape, dim)


def _fused_kernel(idx_ref, xtab_ref, rel_ref, b00_ref, b01_ref, b10_ref, b11_ref,
                  madd_ref, f0_ref, f1_ref,
                  w1_ref, b1_ref, g1_ref, be1_ref,
                  w2_ref, b2_ref, g2_ref, be2_ref,
                  w3_ref, b3_ref, wq0_ref, wq1_ref, wo0_ref, wo1_ref,
                  out0_ref, out1_ref):
    f32 = jnp.float32
    idx = idx_ref[...]
    onehot = (idx == _iota((E, N), 1)).astype(f32)
    xg = jnp.dot(onehot, xtab_ref[...], preferred_element_type=f32, precision=PREC)
    xg0 = xg[:, :F]
    xg1 = [xg[:, F + m * F:F + (m + 1) * F] for m in range(3)]

    rel = rel_ref[...]
    y2s = []
    for p in range(8):
        y = rel * w1_ref[p:p + 1, :] + b1_ref[p:p + 1, :]
        y = jax.nn.relu(_ln(y, g1_ref[p:p + 1, :], be1_ref[p:p + 1, :]))
        y = jnp.dot(y.astype(jnp.bfloat16), w2_ref[p],
                    preferred_element_type=f32) + b2_ref[p:p + 1, :]
        y = jax.nn.relu(_ln(y, g2_ref[p:p + 1, :], be2_ref[p:p + 1, :]))
        y2s.append(y.astype(jnp.bfloat16))

    Ttile = (_iota((F, F * HID), 1) // HID == _iota((F, F * HID), 0)).astype(f32)
    xg0t = jnp.dot(xg0, Ttile, preferred_element_type=f32, precision=PREC)
    xg1t = [jnp.dot(xg1[m], Ttile, preferred_element_type=f32, precision=PREC)
            for m in range(3)]

    def foldsum(prod, fc):
        w = fc
        while w % 2 == 0:
            half = (w // 2) * HID
            prod = prod[:, :half] + prod[:, half:]
            w //= 2
        if w == 3:
            prod = prod[:, :HID] + prod[:, HID:2 * HID] + prod[:, 2 * HID:]
        return prod

    b00 = b00_ref[...]
    b01 = b01_ref[...]
    b10 = b10_ref[...]
    b11 = b11_ref[...]

    kv0 = []
    kv1 = []
    for cv in range(2):
        acc0 = jnp.zeros((E, HID), f32)
        acc1 = [jnp.zeros((E, HID), f32) for _ in range(3)]
        for ip, (di, do) in enumerate(DEG_PAIRS):
            p = cv * 4 + ip
            w3 = w3_ref[:, W3_OFFS[p]:W3_OFFS[p] + W3_WIDTHS[ip]]
            b3 = b3_ref[:, W3_OFFS[p]:W3_OFFS[p] + W3_WIDTHS[ip]]
            y3 = jnp.dot(y2s[p], w3, preferred_element_type=f32) + b3
            if (di, do) == (0, 0):
                acc0 = acc0 + b00 * foldsum(y3 * xg0t, F)
            elif (di, do) == (0, 1):
                Q = foldsum(y3 * xg0t, F)
                for mo in range(3):
                    acc1[mo] = acc1[mo] + b01[:, mo:mo + 1] * Q
            elif (di, do) == (1, 0):
                for mi in range(3):
                    acc0 = acc0 + b10[:, mi:mi + 1] * foldsum(y3 * xg1t[mi], F)
            else:
                P = [[foldsum(y3[:, c * F * HID:(c + 1) * F * HID] * xg1t[mi], F)
                      for mi in range(3)] for c in range(3)]
                for mo in range(3):
                    for c in range(3):
                        for mi in range(3):
                            col = 9 * mo + 3 * mi + c
                            acc1[mo] = acc1[mo] + b11[:, col:col + 1] * P[c][mi]
        kv0.append(acc0)
        kv1.append(acc1)
    K0, V0 = kv0
    K1, V1 = kv1

    scale = DH ** -0.5
    madd = madd_ref[...]
    f0 = f0_ref[...]
    f1 = f1_ref[...]
    q0 = jnp.dot(f0, wq0_ref[...], preferred_element_type=f32, precision=PREC)
    q1 = [jnp.dot(f1[:, m * F:(m + 1) * F], wq1_ref[...], preferred_element_type=f32, precision=PREC)
          for m in range(3)]

    Sh = (_iota((HID, HEADS), 0) // DH == _iota((HID, HEADS), 1)).astype(f32)
    Gn = (_iota((BN, E), 1) // NB == _iota((BN, E), 0)).astype(f32)
    GnT = (_iota((E, BN), 0) // NB == _iota((E, BN), 1)).astype(f32)
    Gk = (_iota((E, NB), 0) % NB == _iota((E, NB), 1)).astype(f32)
    qe0 = jnp.dot(GnT, q0, preferred_element_type=f32, precision=PREC)
    qe1 = [jnp.dot(GnT, q, preferred_element_type=f32, precision=PREC) for q in q1]

    def softmax_full(s):
        sim = jnp.dot(s, Sh, preferred_element_type=f32, precision=PREC) * scale + madd
        cols = []
        for h in range(HEADS):
            mat = jnp.dot(Gn, sim[:, h:h + 1] * Gk, preferred_element_type=f32, precision=PREC)
            mx = mat.max(axis=-1, keepdims=True)
            ex = jnp.exp(mat - mx)
            at = ex / ex.sum(axis=-1, keepdims=True)
            ate = (jnp.dot(GnT, at, preferred_element_type=f32, precision=PREC) * Gk).sum(
                axis=-1, keepdims=True)
            cols.append(jnp.broadcast_to(ate, (E, DH)))
        return jnp.concatenate(cols, axis=1)

    a0 = softmax_full(qe0 * K0)
    o0 = jnp.dot(Gn, a0 * V0, preferred_element_type=f32, precision=PREC)
    out0_ref[...] = jnp.dot(o0, wo0_ref[...], preferred_element_type=f32, precision=PREC)

    s1 = qe1[0] * K1[0] + qe1[1] * K1[1] + qe1[2] * K1[2]
    a1 = softmax_full(s1)
    outs = []
    for m in range(3):
        o1 = jnp.dot(Gn, a1 * V1[m], preferred_element_type=f32, precision=PREC)
        outs.append(jnp.dot(o1, wo1_ref[...], preferred_element_type=f32, precision=PREC))
    out1_ref[...] = jnp.concatenate(outs, axis=1)


def kernel(feat0, feat1, neighbor_indices, neighbor_mask, rel_dist, basis, params):
    B = feat0.shape[0]
    f0 = feat0.reshape(N, F).astype(jnp.float32)
    f1m = feat1.reshape(N, F, 3).transpose(0, 2, 1).reshape(N, 3 * F).astype(jnp.float32)
    xtab = jnp.concatenate([f0, f1m], axis=1)
    idx = neighbor_indices.reshape(N * NB, 1).astype(jnp.int32)
    rel = rel_dist.reshape(N * NB, 1).astype(jnp.float32)
    madd = jnp.where(neighbor_mask.reshape(N * NB, 1), 0.0, -1e30).astype(jnp.float32)
    b00 = basis['0,0'].reshape(N * NB, 1)
    b01 = basis['0,1'].reshape(N * NB, 3)
    b10 = basis['1,0'].reshape(N * NB, 3)
    b11 = basis['1,1'].reshape(N * NB, 27)

    pk, pv = params['to_k'], params['to_v']
    pipes = [pk['0,0'], pk['0,1'], pk['1,0'], pk['1,1'],
             pv['0,0'], pv['0,1'], pv['1,0'], pv['1,1']]

    def permute_w3(p, ip):
        w3, b3 = p['w3'], p['b3']
        if ip == 3:
            w3 = w3.reshape(MID, HID, F, 3).transpose(0, 3, 2, 1).reshape(MID, 1536)
            b3 = b3.reshape(HID, F, 3).transpose(2, 1, 0).reshape(1536)
        else:
            w3 = w3.reshape(MID, HID, F).transpose(0, 2, 1).reshape(MID, 512)
            b3 = b3.reshape(HID, F).transpose(1, 0).reshape(512)
        return w3, b3

    w3b3 = [permute_w3(p, i % 4) for i, p in enumerate(pipes)]
    w1 = jnp.stack([p['w1'].reshape(MID) for p in pipes])
    b1 = jnp.stack([p['b1'] for p in pipes])
    g1 = jnp.stack([p['g1'] for p in pipes])
    be1 = jnp.stack([p['be1'] for p in pipes])
    w2 = jnp.stack([p['w2'] for p in pipes]).astype(jnp.bfloat16)
    b2 = jnp.stack([p['b2'] for p in pipes])
    g2 = jnp.stack([p['g2'] for p in pipes])
    be2 = jnp.stack([p['be2'] for p in pipes])
    w3 = jnp.concatenate([w for w, _ in w3b3], axis=1).astype(jnp.bfloat16)
    b3 = jnp.concatenate([b for _, b in w3b3])[None, :]
    wq0 = params['to_q']['0']
    wq1 = params['to_q']['1']
    wo0 = params['to_out']['0']
    wo1 = params['to_out']['1']

    def blk(shape):
        return pl.BlockSpec(shape, lambda i: (i,) + (0,) * (len(shape) - 1))

    def rep(shape):
        return pl.BlockSpec(shape, lambda i: (0,) * len(shape))

    out0, out1 = pl.pallas_call(
        _fused_kernel,
        grid=(N // BN,),
        in_specs=[
            blk((E, 1)),
            rep((N, 4 * F)),
            blk((E, 1)),
            blk((E, 1)),
            blk((E, 3)),
            blk((E, 3)),
            blk((E, 27)),
            blk((E, 1)),
            blk((BN, F)),
            blk((BN, 3 * F)),
            rep((8, MID)), rep((8, MID)), rep((8, MID)), rep((8, MID)),
            rep((8, MID, MID)),
            rep((8, MID)), rep((8, MID)), rep((8, MID)),
            rep((MID, 6144)), rep((1, 6144)),
            rep((F, HID)), rep((F, HID)), rep((HID, F)), rep((HID, F)),
        ],
        out_specs=[blk((BN, F)), blk((BN, 3 * F))],
        out_shape=[jax.ShapeDtypeStruct((N, F), jnp.float32),
                   jax.ShapeDtypeStruct((N, 3 * F), jnp.float32)],
    )(idx, xtab, rel, b00, b01, b10, b11, madd, f0, f1m,
      w1, b1, g1, be1, w2, b2, g2, be2, w3, b3, wq0, wq1, wo0, wo1)

    out1 = out1.reshape(N, 3, F).transpose(0, 2, 1)
    return out0.reshape(B, N, F, 1), out1.reshape(B, N, F, 3)
